# baseline (speedup 1.0000x reference)
import functools

import jax
import jax.numpy as jnp
from jax import lax
from jax.experimental import pallas as pl
from jax.experimental.pallas import tpu as pltpu
from jax.experimental.pallas import tpu_sc as plsc

N_ROWS = 65536
N_COLS = 512
LANES = 16
NC = 2
NS = 16
NW = NC * NS
ROWS_PER_W = N_ROWS // NW
CHUNK_R = 64
N_CHUNKS = ROWS_PER_W // CHUNK_R
BLKS = N_COLS // LANES


@functools.partial(
    pl.kernel,
    out_type=jax.ShapeDtypeStruct((N_ROWS * N_COLS,), jnp.float32),
    mesh=plsc.VectorSubcoreMesh(core_axis_name="c", subcore_axis_name="s"),
    compiler_params=pltpu.CompilerParams(needs_layout_passes=False),
    scratch_types=[
        pltpu.VMEM((N_COLS,), jnp.int32),
        pltpu.VMEM((CHUNK_R * N_COLS,), jnp.float32),
        pltpu.VMEM((CHUNK_R * N_COLS,), jnp.float32),
    ],
)
def _permute_sc(x_hbm, perm_hbm, out_hbm, perm_v, in_v, out_v):
    wid = lax.axis_index("s") * NC + lax.axis_index("c")
    pltpu.sync_copy(perm_hbm, perm_v)
    row0 = wid * ROWS_PER_W

    def chunk_body(c, carry):
        base = (row0 + c * CHUNK_R) * N_COLS
        pltpu.sync_copy(x_hbm.at[pl.ds(base, CHUNK_R * N_COLS)], in_v)
        for b in range(BLKS):
            idx0 = perm_v[pl.ds(LANES * b, LANES)]

            def row_body(r, idx):
                out_v[pl.ds(r * N_COLS + LANES * b, LANES)] = (
                    plsc.load_gather(in_v, [idx]))
                return idx + N_COLS

            lax.fori_loop(0, CHUNK_R, row_body, idx0, unroll=4)
        pltpu.sync_copy(out_v, out_hbm.at[pl.ds(base, CHUNK_R * N_COLS)])
        return carry

    lax.fori_loop(0, N_CHUNKS, chunk_body, 0)


def kernel(x, perm):
    out_flat = _permute_sc(x.reshape(-1), perm)
    return out_flat.reshape(N_ROWS, N_COLS)


# speedup vs baseline: 1.2359x; 1.2359x over previous
import functools

import jax
import jax.numpy as jnp
from jax import lax
from jax.experimental import pallas as pl
from jax.experimental.pallas import tpu as pltpu
from jax.experimental.pallas import tpu_sc as plsc

N_ROWS = 65536
N_COLS = 512
LANES = 16
NC = 2
NS = 16
NW = NC * NS
ROWS_PER_W = N_ROWS // NW
CHUNK_R = 32
CHUNK_W = CHUNK_R * N_COLS
N_CHUNKS = ROWS_PER_W // CHUNK_R
BLKS = N_COLS // LANES
GRP = 8
N_GRPS = BLKS // GRP


@functools.partial(
    pl.kernel,
    out_type=jax.ShapeDtypeStruct((N_ROWS * N_COLS,), jnp.float32),
    mesh=plsc.VectorSubcoreMesh(core_axis_name="c", subcore_axis_name="s"),
    compiler_params=pltpu.CompilerParams(needs_layout_passes=False),
    scratch_types=[
        pltpu.VMEM((N_COLS,), jnp.int32),
        pltpu.VMEM((CHUNK_W,), jnp.float32),
        pltpu.VMEM((CHUNK_W,), jnp.float32),
        pltpu.VMEM((CHUNK_W,), jnp.float32),
        pltpu.VMEM((CHUNK_W,), jnp.float32),
        pltpu.SemaphoreType.DMA,
        pltpu.SemaphoreType.DMA,
        pltpu.SemaphoreType.DMA,
        pltpu.SemaphoreType.DMA,
    ],
)
def _permute_sc(x_hbm, perm_hbm, out_hbm, perm_v,
                in_v0, in_v1, out_v0, out_v1,
                sem_i0, sem_i1, sem_o0, sem_o1):
    wid = lax.axis_index("s") * NC + lax.axis_index("c")
    pltpu.sync_copy(perm_hbm, perm_v)
    base0 = wid * ROWS_PER_W * N_COLS

    bufs = ((in_v0, out_v0, sem_i0, sem_o0),
            (in_v1, out_v1, sem_i1, sem_o1))

    def start_in(chunk, in_v, sem):
        pltpu.make_async_copy(
            x_hbm.at[pl.ds(base0 + chunk * CHUNK_W, CHUNK_W)], in_v, sem
        ).start()

    start_in(0, in_v0, sem_i0)
    start_in(1, in_v1, sem_i1)

    def permute_chunk(in_v, out_v):
        for g in range(N_GRPS):
            idxs = [perm_v[pl.ds(LANES * (g * GRP + k), LANES)]
                    for k in range(GRP)]

            def row_body(r, carry):
                src = in_v.at[pl.ds(r * N_COLS, N_COLS)]
                obase = r * N_COLS + LANES * g * GRP
                for k in range(GRP):
                    out_v[pl.ds(obase + LANES * k, LANES)] = (
                        plsc.load_gather(src, [idxs[k]]))
                return carry

            lax.fori_loop(0, CHUNK_R, row_body, 0, unroll=2)

    def pair_body(i, carry):
        for p in range(2):
            in_v, out_v, sem_i, sem_o = bufs[p]
            chunk = 2 * i + p
            pltpu.make_async_copy(
                x_hbm.at[pl.ds(0, CHUNK_W)], in_v, sem_i).wait()

            @pl.when(chunk >= 2)
            def _():
                pltpu.make_async_copy(
                    out_v, out_hbm.at[pl.ds(0, CHUNK_W)], sem_o).wait()

            permute_chunk(in_v, out_v)
            pltpu.make_async_copy(
                out_v, out_hbm.at[pl.ds(base0 + chunk * CHUNK_W, CHUNK_W)],
                sem_o).start()

            @pl.when(chunk + 2 < N_CHUNKS)
            def _():
                start_in(chunk + 2, in_v, sem_i)
        return carry

    lax.fori_loop(0, N_CHUNKS // 2, pair_body, 0)

    pltpu.make_async_copy(out_v0, out_hbm.at[pl.ds(0, CHUNK_W)], sem_o0).wait()
    pltpu.make_async_copy(out_v1, out_hbm.at[pl.ds(0, CHUNK_W)], sem_o1).wait()


def kernel(x, perm):
    out_flat = _permute_sc(x.reshape(-1), perm)
    return out_flat.reshape(N_ROWS, N_COLS)


# speedup vs baseline: 1.8582x; 1.5035x over previous
import functools

import jax
import jax.numpy as jnp
from jax import lax
from jax.experimental import pallas as pl
from jax.experimental.pallas import tpu as pltpu
from jax.experimental.pallas import tpu_sc as plsc

N_ROWS = 65536
N_COLS = 512
LANES = 16
NC = 2
NS = 16
NW = NC * NS
ROWS_PER_W = N_ROWS // NW
CHUNK_R = 32
CHUNK_W = CHUNK_R * N_COLS
N_CHUNKS = ROWS_PER_W // CHUNK_R
BLKS = N_COLS // LANES
GRP = 8
N_GRPS = BLKS // GRP


@functools.partial(
    pl.kernel,
    out_type=jax.ShapeDtypeStruct((N_ROWS * N_COLS,), jnp.float32),
    mesh=plsc.VectorSubcoreMesh(core_axis_name="c", subcore_axis_name="s"),
    compiler_params=pltpu.CompilerParams(needs_layout_passes=False),
    scratch_types=[
        pltpu.VMEM((N_COLS,), jnp.int32),
        pltpu.VMEM((CHUNK_W,), jnp.float32),
        pltpu.VMEM((CHUNK_W,), jnp.float32),
        pltpu.VMEM((CHUNK_W,), jnp.float32),
        pltpu.VMEM((CHUNK_W,), jnp.float32),
        pltpu.SemaphoreType.DMA,
        pltpu.SemaphoreType.DMA,
        pltpu.SemaphoreType.DMA,
        pltpu.SemaphoreType.DMA,
    ],
)
def _permute_sc(x_hbm, perm_hbm, out_hbm, perm_v,
                in_v0, in_v1, out_v0, out_v1,
                sem_i0, sem_i1, sem_o0, sem_o1):
    wid = lax.axis_index("s") * NC + lax.axis_index("c")
    pltpu.sync_copy(perm_hbm, perm_v)
    base0 = wid * ROWS_PER_W * N_COLS

    bufs = ((in_v0, out_v0, sem_i0, sem_o0),
            (in_v1, out_v1, sem_i1, sem_o1))

    def start_in(chunk, in_v, sem):
        pltpu.make_async_copy(
            x_hbm.at[pl.ds(base0 + chunk * CHUNK_W, CHUNK_W)], in_v, sem
        ).start()

    start_in(0, in_v0, sem_i0)
    start_in(1, in_v1, sem_i1)

    def permute_chunk(in_v, out_v):
        for g in range(N_GRPS):
            idxs = [perm_v[pl.ds(LANES * (g * GRP + k), LANES)]
                    for k in range(GRP)]

            @plsc.parallel_loop(0, CHUNK_R, unroll=4)
            def _(r):
                src = in_v.at[pl.ds(r * N_COLS, N_COLS)]
                obase = r * N_COLS + LANES * g * GRP
                for k in range(GRP):
                    out_v[pl.ds(obase + LANES * k, LANES)] = (
                        plsc.load_gather(src, [idxs[k]]))

    def pair_body(i, carry):
        for p in range(2):
            in_v, out_v, sem_i, sem_o = bufs[p]
            chunk = 2 * i + p
            pltpu.make_async_copy(
                x_hbm.at[pl.ds(0, CHUNK_W)], in_v, sem_i).wait()

            @pl.when(chunk >= 2)
            def _():
                pltpu.make_async_copy(
                    out_v, out_hbm.at[pl.ds(0, CHUNK_W)], sem_o).wait()

            permute_chunk(in_v, out_v)
            pltpu.make_async_copy(
                out_v, out_hbm.at[pl.ds(base0 + chunk * CHUNK_W, CHUNK_W)],
                sem_o).start()

            @pl.when(chunk + 2 < N_CHUNKS)
            def _():
                start_in(chunk + 2, in_v, sem_i)
        return carry

    lax.fori_loop(0, N_CHUNKS // 2, pair_body, 0)

    pltpu.make_async_copy(out_v0, out_hbm.at[pl.ds(0, CHUNK_W)], sem_o0).wait()
    pltpu.make_async_copy(out_v1, out_hbm.at[pl.ds(0, CHUNK_W)], sem_o1).wait()


def kernel(x, perm):
    out_flat = _permute_sc(x.reshape(-1), perm)
    return out_flat.reshape(N_ROWS, N_COLS)


# speedup vs baseline: 5.3607x; 2.8849x over previous
import functools

import jax
import jax.numpy as jnp
from jax import lax
from jax.experimental import pallas as pl
from jax.experimental.pallas import tpu as pltpu
from jax.experimental.pallas import tpu_sc as plsc

N_ROWS = 65536
N_COLS = 512
LANES = 16
NC = 2
NS = 16
NW = NC * NS
ROWS_PER_W = N_ROWS // NW
CHUNK_R = 32
N_CHUNKS = ROWS_PER_W // CHUNK_R
BLKS = N_COLS // LANES
GRP = 8
N_GRPS = BLKS // GRP


@functools.partial(
    pl.kernel,
    out_type=jax.ShapeDtypeStruct((N_ROWS, N_COLS), jnp.float32),
    mesh=plsc.VectorSubcoreMesh(core_axis_name="c", subcore_axis_name="s"),
    compiler_params=pltpu.CompilerParams(needs_layout_passes=False),
    scratch_types=[
        pltpu.VMEM((N_COLS,), jnp.int32),
        pltpu.VMEM((CHUNK_R, N_COLS), jnp.float32),
        pltpu.VMEM((CHUNK_R, N_COLS), jnp.float32),
        pltpu.VMEM((CHUNK_R, N_COLS), jnp.float32),
        pltpu.VMEM((CHUNK_R, N_COLS), jnp.float32),
        pltpu.SemaphoreType.DMA,
        pltpu.SemaphoreType.DMA,
        pltpu.SemaphoreType.DMA,
        pltpu.SemaphoreType.DMA,
    ],
)
def _permute_sc(x_hbm, perm_hbm, out_hbm, perm_v,
                in_v0, in_v1, out_v0, out_v1,
                sem_i0, sem_i1, sem_o0, sem_o1):
    wid = lax.axis_index("s") * NC + lax.axis_index("c")
    pltpu.sync_copy(perm_hbm, perm_v)
    row0 = wid * ROWS_PER_W

    bufs = ((in_v0, out_v0, sem_i0, sem_o0),
            (in_v1, out_v1, sem_i1, sem_o1))

    def start_in(chunk, in_v, sem):
        pltpu.make_async_copy(
            x_hbm.at[pl.ds(row0 + chunk * CHUNK_R, CHUNK_R), :], in_v, sem
        ).start()

    start_in(0, in_v0, sem_i0)
    start_in(1, in_v1, sem_i1)

    def permute_chunk(in_v, out_v):
        for g in range(N_GRPS):
            idxs = [perm_v[pl.ds(LANES * (g * GRP + k), LANES)]
                    for k in range(GRP)]

            @plsc.parallel_loop(0, CHUNK_R, unroll=4)
            def _(r):
                row_idx = jnp.full((LANES,), r, dtype=jnp.int32)
                for k in range(GRP):
                    out_v[r, pl.ds(LANES * (g * GRP + k), LANES)] = (
                        plsc.load_gather(in_v, [row_idx, idxs[k]]))

    def pair_body(i, carry):
        for p in range(2):
            in_v, out_v, sem_i, sem_o = bufs[p]
            chunk = 2 * i + p
            pltpu.make_async_copy(
                x_hbm.at[pl.ds(0, CHUNK_R), :], in_v, sem_i).wait()

            @pl.when(chunk >= 2)
            def _():
                pltpu.make_async_copy(
                    out_v, out_hbm.at[pl.ds(0, CHUNK_R), :], sem_o).wait()

            permute_chunk(in_v, out_v)
            pltpu.make_async_copy(
                out_v, out_hbm.at[pl.ds(row0 + chunk * CHUNK_R, CHUNK_R), :],
                sem_o).start()

            @pl.when(chunk + 2 < N_CHUNKS)
            def _():
                start_in(chunk + 2, in_v, sem_i)
        return carry

    lax.fori_loop(0, N_CHUNKS // 2, pair_body, 0)

    pltpu.make_async_copy(
        out_v0, out_hbm.at[pl.ds(0, CHUNK_R), :], sem_o0).wait()
    pltpu.make_async_copy(
        out_v1, out_hbm.at[pl.ds(0, CHUNK_R), :], sem_o1).wait()


def kernel(x, perm):
    return _permute_sc(x, perm)
